# Initial kernel scaffold; baseline (speedup 1.0000x reference)
import jax
import jax.numpy as jnp
from jax.experimental import pallas as pl


def kernel(x, h0, W_ih, W_hh):
    raise NotImplementedError("write your pallas kernel here")


# speedup vs baseline: 9.3587x; 9.3587x over previous
import functools

import jax
import jax.numpy as jnp
from jax.experimental import pallas as pl
from jax.experimental.pallas import tpu as pltpu


def _rnn_block_kernel(x_ref, h0_ref, wih_t_ref, whh_t_ref, out_ref, hn_ref,
                      h_s, xw_s, *, tb_steps):
    t_idx = pl.program_id(1)

    @pl.when(t_idx == 0)
    def _():
        h_s[...] = h0_ref[...]

    bblk = x_ref.shape[0]
    h_dim = whh_t_ref.shape[1]

    x2 = x_ref[...].reshape(bblk * tb_steps, x_ref.shape[2])
    xw = jnp.dot(x2, wih_t_ref[...], preferred_element_type=jnp.float32)
    xw_s[...] = xw.reshape(bblk, tb_steps, h_dim)

    whh_t = whh_t_ref[...]

    def body(i, h):
        h_new = jnp.maximum(
            xw_s[:, i, :] + jnp.dot(h, whh_t, preferred_element_type=jnp.float32),
            0.0,
        )
        out_ref[:, i, :] = h_new
        return h_new

    h = jax.lax.fori_loop(0, tb_steps, body, h_s[...])
    h_s[...] = h
    hn_ref[...] = h


def kernel(x, h0, W_ih, W_hh):
    B, T, I = x.shape
    H = W_hh.shape[0]
    n_cores = 2 if B % 16 == 0 else 1
    bblk = B // n_cores
    tb = 128 if T % 128 == 0 else T
    nt = T // tb

    wih_t = W_ih.T
    whh_t = W_hh.T
    h0_2d = h0[0]

    out, h_n = pl.pallas_call(
        functools.partial(_rnn_block_kernel, tb_steps=tb),
        out_shape=(
            jax.ShapeDtypeStruct((B, T, H), x.dtype),
            jax.ShapeDtypeStruct((B, H), x.dtype),
        ),
        grid=(n_cores, nt),
        in_specs=[
            pl.BlockSpec((bblk, tb, I), lambda b, t: (b, t, 0)),
            pl.BlockSpec((bblk, H), lambda b, t: (b, 0)),
            pl.BlockSpec((I, H), lambda b, t: (0, 0)),
            pl.BlockSpec((H, H), lambda b, t: (0, 0)),
        ],
        out_specs=(
            pl.BlockSpec((bblk, tb, H), lambda b, t: (b, t, 0)),
            pl.BlockSpec((bblk, H), lambda b, t: (b, 0)),
        ),
        scratch_shapes=[
            pltpu.VMEM((bblk, H), jnp.float32),
            pltpu.VMEM((bblk, tb, H), jnp.float32),
        ],
        compiler_params=pltpu.CompilerParams(
            dimension_semantics=("parallel", "arbitrary"),
        ),
        name="rnn_relu_fused",
    )(x, h0_2d, wih_t, whh_t)
    return out, h_n[None]


# speedup vs baseline: 16.0906x; 1.7193x over previous
import functools

import jax
import jax.numpy as jnp
from jax.experimental import pallas as pl
from jax.experimental.pallas import tpu as pltpu


def _rnn_block_kernel(x_ref, h0_ref, wih_t_ref, whh_t_ref, out_ref, hn_ref,
                      h_s, xw_s, *, tb_steps):
    t_idx = pl.program_id(0)

    @pl.when(t_idx == 0)
    def _():
        h_s[...] = h0_ref[...]

    b = x_ref.shape[0]
    h_dim = whh_t_ref.shape[1]

    x2 = x_ref[...].reshape(b * tb_steps, x_ref.shape[2])
    xw = jnp.dot(x2, wih_t_ref[...], preferred_element_type=jnp.float32)
    xw_s[...] = xw.reshape(b, tb_steps, h_dim)

    whh_t = whh_t_ref[...]

    def body(i, h):
        h_new = jnp.maximum(
            xw_s[:, i, :] + jnp.dot(h, whh_t, preferred_element_type=jnp.float32),
            0.0,
        )
        out_ref[:, i, :] = h_new
        return h_new

    h = jax.lax.fori_loop(0, tb_steps, body, h_s[...])
    h_s[...] = h
    hn_ref[...] = h


def kernel(x, h0, W_ih, W_hh):
    B, T, I = x.shape
    H = W_hh.shape[0]
    tb = 128 if T % 128 == 0 else T
    nt = T // tb

    wih_t = W_ih.T
    whh_t = W_hh.T
    h0_2d = h0[0]

    out, h_n = pl.pallas_call(
        functools.partial(_rnn_block_kernel, tb_steps=tb),
        out_shape=(
            jax.ShapeDtypeStruct((B, T, H), x.dtype),
            jax.ShapeDtypeStruct((B, H), x.dtype),
        ),
        grid=(nt,),
        in_specs=[
            pl.BlockSpec((B, tb, I), lambda t: (0, t, 0)),
            pl.BlockSpec((B, H), lambda t: (0, 0)),
            pl.BlockSpec((I, H), lambda t: (0, 0)),
            pl.BlockSpec((H, H), lambda t: (0, 0)),
        ],
        out_specs=(
            pl.BlockSpec((B, tb, H), lambda t: (0, t, 0)),
            pl.BlockSpec((B, H), lambda t: (0, 0)),
        ),
        scratch_shapes=[
            pltpu.VMEM((B, H), jnp.float32),
            pltpu.VMEM((B, tb, H), jnp.float32),
        ],
        compiler_params=pltpu.CompilerParams(
            dimension_semantics=("arbitrary",),
        ),
        name="rnn_relu_fused",
    )(x, h0_2d, wih_t, whh_t)
    return out, h_n[None]


# speedup vs baseline: 18.3344x; 1.1394x over previous
import functools

import jax
import jax.numpy as jnp
from jax.experimental import pallas as pl
from jax.experimental.pallas import tpu as pltpu


def _rnn_block_kernel(x_ref, h0_ref, wih_t_ref, whh_t_ref, out_ref, hn_ref,
                      h_s, xw_s, *, tb_steps):
    t_idx = pl.program_id(0)

    @pl.when(t_idx == 0)
    def _():
        h_s[...] = h0_ref[...]

    b = x_ref.shape[0]
    h_dim = whh_t_ref.shape[1]

    x2 = x_ref[...].reshape(b * tb_steps, x_ref.shape[2])
    xw = jnp.dot(x2, wih_t_ref[...], preferred_element_type=jnp.float32)
    xw_s[...] = xw.reshape(b, tb_steps, h_dim)

    whh_t = whh_t_ref[...]

    def body(i, h):
        h_new = jnp.maximum(
            xw_s[:, i, :] + jnp.dot(h, whh_t, preferred_element_type=jnp.float32),
            0.0,
        )
        out_ref[:, i, :] = h_new
        return h_new

    h = jax.lax.fori_loop(0, tb_steps, body, h_s[...], unroll=8)
    h_s[...] = h
    hn_ref[...] = h


def kernel(x, h0, W_ih, W_hh):
    B, T, I = x.shape
    H = W_hh.shape[0]
    tb = 128 if T % 128 == 0 else T
    nt = T // tb

    wih_t = W_ih.T
    whh_t = W_hh.T
    h0_2d = h0[0]

    out, h_n = pl.pallas_call(
        functools.partial(_rnn_block_kernel, tb_steps=tb),
        out_shape=(
            jax.ShapeDtypeStruct((B, T, H), x.dtype),
            jax.ShapeDtypeStruct((B, H), x.dtype),
        ),
        grid=(nt,),
        in_specs=[
            pl.BlockSpec((B, tb, I), lambda t: (0, t, 0)),
            pl.BlockSpec((B, H), lambda t: (0, 0)),
            pl.BlockSpec((I, H), lambda t: (0, 0)),
            pl.BlockSpec((H, H), lambda t: (0, 0)),
        ],
        out_specs=(
            pl.BlockSpec((B, tb, H), lambda t: (0, t, 0)),
            pl.BlockSpec((B, H), lambda t: (0, 0)),
        ),
        scratch_shapes=[
            pltpu.VMEM((B, H), jnp.float32),
            pltpu.VMEM((B, tb, H), jnp.float32),
        ],
        compiler_params=pltpu.CompilerParams(
            dimension_semantics=("arbitrary",),
        ),
        name="rnn_relu_fused",
    )(x, h0_2d, wih_t, whh_t)
    return out, h_n[None]


# speedup vs baseline: 22.3665x; 1.2199x over previous
import functools

import jax
import jax.numpy as jnp
from jax.experimental import pallas as pl
from jax.experimental.pallas import tpu as pltpu


def _rnn_block_kernel(x_ref, h0_ref, wih_t_ref, whh_t_ref, out_ref, hn_ref,
                      h_s, xw_s, *, tb_steps, bc):
    t_idx = pl.program_id(0)

    @pl.when(t_idx == 0)
    def _():
        h_s[...] = h0_ref[...]

    b = x_ref.shape[0]
    i_dim = x_ref.shape[2]
    h_dim = whh_t_ref.shape[1]
    half = h_dim // 2
    n_chunks = b // bc
    m_rows = bc * tb_steps

    for mxu in range(2):
        for kt in range(2):
            pltpu.matmul_push_rhs(
                wih_t_ref[kt * 256:(kt + 1) * 256, mxu * half:mxu * half + 256],
                staging_register=kt, mxu_index=mxu)

    for mc in range(n_chunks):
        xc = x_ref[mc * bc:(mc + 1) * bc].reshape(m_rows, i_dim)
        addr = (mc % 2) * (m_rows // 4)
        for mxu in range(2):
            pltpu.matmul_acc_lhs(addr, xc[:, :256], mxu_index=mxu,
                                 load_staged_rhs=0)
            pltpu.matmul_acc_lhs(addr, xc[:, 256:], mxu_index=mxu,
                                 load_staged_rhs=1)
        for mxu in range(2):
            y = pltpu.matmul_pop(addr, (m_rows, 256), jnp.float32,
                                 mxu_index=mxu)
            xw_s[mc * bc:(mc + 1) * bc, :, mxu * half:mxu * half + 256] = (
                y.reshape(bc, tb_steps, 256))

    for mxu in range(2):
        for kt in range(2):
            pltpu.matmul_push_rhs(
                whh_t_ref[kt * 256:(kt + 1) * 256, mxu * half:mxu * half + 256],
                staging_register=kt, mxu_index=mxu)

    def body(t, h):
        ha = h[:, :256]
        hb = h[:, 256:]
        for mxu in range(2):
            pltpu.matmul_acc_lhs(0, ha, mxu_index=mxu, load_staged_rhs=0)
            pltpu.matmul_acc_lhs(0, hb, mxu_index=mxu, load_staged_rhs=1)
        y0 = pltpu.matmul_pop(0, (b, 256), jnp.float32, mxu_index=0)
        y1 = pltpu.matmul_pop(0, (b, 256), jnp.float32, mxu_index=1)
        y = jnp.concatenate([y0, y1], axis=1)
        h_new = jnp.maximum(xw_s[:, t, :] + y, 0.0)
        out_ref[:, t, :] = h_new
        return h_new

    h = jax.lax.fori_loop(0, tb_steps, body, h_s[...], unroll=4)
    h_s[...] = h
    hn_ref[...] = h


def kernel(x, h0, W_ih, W_hh):
    B, T, I = x.shape
    H = W_hh.shape[0]
    tb = 128 if T % 128 == 0 else T
    nt = T // tb
    bc = 4 if B % 8 == 0 else B

    wih_t = W_ih.T
    whh_t = W_hh.T
    h0_2d = h0[0]

    out, h_n = pl.pallas_call(
        functools.partial(_rnn_block_kernel, tb_steps=tb, bc=bc),
        out_shape=(
            jax.ShapeDtypeStruct((B, T, H), x.dtype),
            jax.ShapeDtypeStruct((B, H), x.dtype),
        ),
        grid=(nt,),
        in_specs=[
            pl.BlockSpec((B, tb, I), lambda t: (0, t, 0)),
            pl.BlockSpec((B, H), lambda t: (0, 0)),
            pl.BlockSpec((I, H), lambda t: (0, 0)),
            pl.BlockSpec((H, H), lambda t: (0, 0)),
        ],
        out_specs=(
            pl.BlockSpec((B, tb, H), lambda t: (0, t, 0)),
            pl.BlockSpec((B, H), lambda t: (0, 0)),
        ),
        scratch_shapes=[
            pltpu.VMEM((B, H), jnp.float32),
            pltpu.VMEM((B, tb, H), jnp.float32),
        ],
        compiler_params=pltpu.CompilerParams(
            dimension_semantics=("arbitrary",),
            vmem_limit_bytes=56 * 1024 * 1024,
        ),
        name="rnn_relu_xmxu",
    )(x, h0_2d, wih_t, whh_t)
    return out, h_n[None]
